# Initial kernel scaffold; baseline (speedup 1.0000x reference)
import jax
import jax.numpy as jnp
from jax.experimental import pallas as pl


def kernel(input, target):
    raise NotImplementedError("write your pallas kernel here")


# speedup vs baseline: 7.2713x; 7.2713x over previous
import jax
import jax.numpy as jnp
from jax import lax
from jax.experimental import pallas as pl
from jax.experimental.pallas import tpu as pltpu

_R = 512
_C = 8192
_BLK_R = 8
_GRID = _R // _BLK_R
_NHI = 32
_NLO = 128


def _skey(x):
    y = lax.bitcast_convert_type(x, jnp.int32)
    m = y >> 31
    return y ^ (m & jnp.int32(0x7FFFFFFF))


def _softplus(x):
    return jnp.maximum(x, 0.0) + jnp.log1p(jnp.exp(-jnp.abs(x)))


def _hist_accum(hist_ref, hi, lo, mask):
    for r in range(_BLK_R):
        hrow = hi[r:r + 1, :]
        lrow = lo[r:r + 1, :]
        mrow = mask[r:r + 1, :]
        ihi = lax.broadcasted_iota(jnp.int32, (_NHI, _C), 0)
        ilo = lax.broadcasted_iota(jnp.int32, (_NLO, _C), 0)
        a_hi = ((ihi == hrow) & mrow).astype(jnp.bfloat16)
        a_lo = (ilo == lrow).astype(jnp.bfloat16)
        hist_ref[...] += lax.dot_general(
            a_hi, a_lo, (((1,), (1,)), ((), ())),
            preferred_element_type=jnp.float32)


def _select(hist, k):
    rowsum = jnp.sum(hist, axis=1, keepdims=True)
    i0 = lax.broadcasted_iota(jnp.int32, (_NHI, _NHI), 0)
    i1 = lax.broadcasted_iota(jnp.int32, (_NHI, _NHI), 1)
    m32 = (i1 > i0).astype(jnp.float32)
    rows_above = lax.dot_general(m32, rowsum, (((1,), (0,)), ((), ())),
                                 preferred_element_type=jnp.float32)
    j0 = lax.broadcasted_iota(jnp.int32, (_NLO, _NLO), 0)
    j1 = lax.broadcasted_iota(jnp.int32, (_NLO, _NLO), 1)
    t128 = (j0 > j1).astype(jnp.float32)
    row_suffix = lax.dot_general(hist, t128, (((1,), (0,)), ((), ())),
                                 preferred_element_type=jnp.float32)
    cum_above = rows_above + row_suffix
    sel = ((cum_above < k) & (cum_above + hist >= k)).astype(jnp.float32)
    bhi = lax.broadcasted_iota(jnp.int32, (_NHI, _NLO), 0).astype(jnp.float32)
    blo = lax.broadcasted_iota(jnp.int32, (_NHI, _NLO), 1).astype(jnp.float32)
    binf = jnp.sum(sel * (bhi * _NLO + blo))
    c_above = jnp.sum(sel * cum_above)
    found = jnp.sum(sel)
    return binf, c_above, found


def _getcol(selv, i):
    r0 = lax.broadcasted_iota(jnp.int32, (8, 128), 0) == 0
    ci = lax.broadcasted_iota(jnp.int32, (8, 128), 1) == i
    return jnp.sum(jnp.where(r0 & ci, selv, 0.0))


def _putrow0(vals):
    r0 = lax.broadcasted_iota(jnp.int32, (8, 128), 0) == 0
    col = lax.broadcasted_iota(jnp.int32, (8, 128), 1)
    out = jnp.zeros((8, 128), jnp.float32)
    for i, v in enumerate(vals):
        out = jnp.where(r0 & (col == i), v, out)
    return out


def _k1_body(x_ref, t_ref, o_ref, hist, acc):
    pid = pl.program_id(0)

    @pl.when(pid == 0)
    def _init():
        hist[...] = jnp.zeros((_NHI, _NLO), jnp.float32)
        acc[0] = 0.0
        acc[1] = 0.0
        o_ref[...] = jnp.zeros((8, 128), jnp.float32)

    x = x_ref[...]
    t = t_ref[...]
    pos = t > 0
    neg = t == 0
    bce1 = jnp.maximum(x, 0.0) - x + jnp.log1p(jnp.exp(-jnp.abs(x)))
    acc[0] += jnp.sum(pos.astype(jnp.float32))
    acc[1] += jnp.sum(jnp.where(pos, bce1, 0.0))

    sk = _skey(x)
    bin12 = (sk >> 20) + 2048
    hi = bin12 >> 7
    lo = bin12 & 127
    _hist_accum(hist, hi, lo, neg)

    @pl.when(pid == _GRID - 1)
    def _fin():
        pos_cnt = acc[0]
        kf = jnp.floor(pos_cnt * 0.5)
        binf, c_above, found = _select(hist[...], kf)
        o_ref[...] = _putrow0([binf, c_above, kf, pos_cnt, acc[1], found])


def _k2_body(x_ref, t_ref, s_ref, o_ref, hist):
    pid = pl.program_id(0)

    @pl.when(pid == 0)
    def _init():
        hist[...] = jnp.zeros((_NHI, _NLO), jnp.float32)
        o_ref[...] = jnp.zeros((8, 128), jnp.float32)

    sv = s_ref[...]
    binf1 = _getcol(sv, 0)
    x = x_ref[...]
    t = t_ref[...]
    neg = t == 0
    sk = _skey(x)
    bin12f = ((sk >> 20) + 2048).astype(jnp.float32)
    inb1 = neg & (bin12f == binf1)
    bin2 = (sk >> 8) & 4095
    hi = bin2 >> 7
    lo = bin2 & 127
    _hist_accum(hist, hi, lo, inb1)

    @pl.when(pid == _GRID - 1)
    def _fin():
        c_above1 = _getcol(sv, 1)
        kf = _getcol(sv, 2)
        pos_cnt = _getcol(sv, 3)
        pos_sum = _getcol(sv, 4)
        found1 = _getcol(sv, 5)
        k2 = kf - c_above1
        b2f, c_above2, found2 = _select(hist[...], k2)
        valid = (found1 > 0.5) & (found2 > 0.5)
        s24 = (binf1 - 2048.0) * 4096.0 + b2f
        extra = kf - c_above1 - c_above2
        kzero = kf < 0.5
        s24 = jnp.where(valid, s24,
                        jnp.where(kzero, 8388608.0, -8388609.0))
        extra = jnp.where(valid, extra, 0.0)
        o_ref[...] = _putrow0([s24, extra, kf, pos_cnt, pos_sum])


def _k3_body(x_ref, t_ref, s_ref, o_ref, acc):
    pid = pl.program_id(0)

    @pl.when(pid == 0)
    def _init():
        acc[0] = 0.0
        o_ref[...] = jnp.zeros((8, 128), jnp.float32)

    sv = s_ref[...]
    s24 = _getcol(sv, 0)
    x = x_ref[...]
    t = t_ref[...]
    neg = t == 0
    sk = _skey(x)
    sk24f = (sk >> 8).astype(jnp.float32)
    cond = neg & (sk24f > s24)
    acc[0] += jnp.sum(jnp.where(cond, _softplus(x), 0.0))

    @pl.when(pid == _GRID - 1)
    def _fin():
        extra = _getcol(sv, 1)
        kf = _getcol(sv, 2)
        pos_cnt = _getcol(sv, 3)
        pos_sum = _getcol(sv, 4)
        s24c = jnp.clip(jnp.full((8, 128), s24), -8388608.0, 8388607.0)
        mid = s24c.astype(jnp.int32) * 256 + 128
        u = jnp.where(mid >= 0, mid, mid ^ jnp.int32(0x7FFFFFFF))
        v = lax.bitcast_convert_type(u, jnp.float32)
        sp_v = jnp.mean(_softplus(v))
        neg_sum = acc[0] + extra * sp_v
        loss = (pos_sum + neg_sum) / (pos_cnt + kf)
        o_ref[...] = jnp.full((8, 128), loss)


def _data_specs():
    xspec = pl.BlockSpec((_BLK_R, _C), lambda i: (i, 0))
    return xspec


def kernel(input, target):
    xv = input.reshape(_R, _C)
    tv = target.reshape(_R, _C).astype(jnp.int32)
    xspec = pl.BlockSpec((_BLK_R, _C), lambda i: (i, 0))
    tspec = pl.BlockSpec((_BLK_R, _C), lambda i: (i, 0))
    sspec = pl.BlockSpec((8, 128), lambda i: (0, 0))
    oshape = jax.ShapeDtypeStruct((8, 128), jnp.float32)

    sel1 = pl.pallas_call(
        _k1_body,
        grid=(_GRID,),
        in_specs=[xspec, tspec],
        out_specs=sspec,
        out_shape=oshape,
        scratch_shapes=[pltpu.VMEM((_NHI, _NLO), jnp.float32),
                        pltpu.SMEM((2,), jnp.float32)],
    )(xv, tv)

    sel2 = pl.pallas_call(
        _k2_body,
        grid=(_GRID,),
        in_specs=[xspec, tspec, sspec],
        out_specs=sspec,
        out_shape=oshape,
        scratch_shapes=[pltpu.VMEM((_NHI, _NLO), jnp.float32)],
    )(xv, tv, sel1)

    out = pl.pallas_call(
        _k3_body,
        grid=(_GRID,),
        in_specs=[xspec, tspec, sspec],
        out_specs=sspec,
        out_shape=oshape,
        scratch_shapes=[pltpu.SMEM((2,), jnp.float32)],
    )(xv, tv, sel2)

    return out[0, 0]


# speedup vs baseline: 14.4772x; 1.9910x over previous
import functools

import jax
import jax.numpy as jnp
from jax import lax
from jax.experimental import pallas as pl
from jax.experimental.pallas import tpu as pltpu
from jax.experimental.pallas import tpu_sc as plsc

_N = 128 * 32768
_NW = 32
_PW = _N // _NW
_CH = 4096
_R = 512
_C = 8192
_BLK_R = 8
_GRID = _R // _BLK_R


def _skey(x):
    y = lax.bitcast_convert_type(x, jnp.int32)
    m = y >> 31
    return y ^ (m & jnp.int32(0x7FFFFFFF))


def _softplus(x):
    return jnp.maximum(x, 0.0) + jnp.log1p(jnp.exp(-jnp.abs(x)))


def _sc_l1_body(x_hbm, t_hbm, zz_hbm,
                hist_out, pos_out,
                xb, tb, histv, posv):
    cid = lax.axis_index("c")
    sid = lax.axis_index("s")
    wid = sid * 2 + cid
    pltpu.sync_copy(zz_hbm, histv)

    lane = lax.broadcasted_iota(jnp.int32, (16,), 0)
    ones = jnp.full((16,), 1, jnp.int32)

    def chunk_body(c, pos_acc):
        base = wid * _PW + c * _CH
        pltpu.sync_copy(x_hbm.at[pl.ds(base, _CH)], xb)
        pltpu.sync_copy(t_hbm.at[pl.ds(base, _CH)], tb)

        def vec_body(i, acc):
            x = xb[pl.ds(i * 16, 16)]
            t = tb[pl.ds(i * 16, 16)]
            neg = t == 0
            acc = acc + jnp.where(t > 0, 1, 0).astype(jnp.int32)
            y = plsc.bitcast(x, jnp.int32)
            sk = y ^ ((y >> 31) & jnp.int32(0x7FFFFFFF))
            b = (sk >> 21) + 1024
            idx = b * 16 + lane
            plsc.addupdate_scatter(histv, [idx >> 8, idx & 255], ones,
                                   mask=neg)
            return acc

        return lax.fori_loop(0, _CH // 16, vec_body, pos_acc)

    posvec = lax.fori_loop(0, _PW // _CH, chunk_body,
                           jnp.zeros((16,), jnp.int32))
    pltpu.sync_copy(histv, hist_out.at[wid])
    posv[...] = posvec
    pltpu.sync_copy(posv, pos_out.at[wid])


def _sc_l2_body(x_hbm, t_hbm, zz_hbm, b1_hbm,
                hist_out,
                xb, tb, histv, b1v):
    cid = lax.axis_index("c")
    sid = lax.axis_index("s")
    wid = sid * 2 + cid
    pltpu.sync_copy(zz_hbm, histv)
    pltpu.sync_copy(b1_hbm.at[0], b1v)

    lane = lax.broadcasted_iota(jnp.int32, (16,), 0)
    ones = jnp.full((16,), 1, jnp.int32)

    def chunk_body(c, carry):
        base = wid * _PW + c * _CH
        pltpu.sync_copy(x_hbm.at[pl.ds(base, _CH)], xb)
        pltpu.sync_copy(t_hbm.at[pl.ds(base, _CH)], tb)
        b1 = b1v[pl.ds(0, 16)]

        def vec_body(i, acc):
            x = xb[pl.ds(i * 16, 16)]
            t = tb[pl.ds(i * 16, 16)]
            y = plsc.bitcast(x, jnp.int32)
            sk = y ^ ((y >> 31) & jnp.int32(0x7FFFFFFF))
            b = (sk >> 21) + 1024
            inb = (t == 0) & (b == b1)
            b2 = (sk >> 9) & 4095
            idx = b2 * 16 + lane
            plsc.addupdate_scatter(histv, [idx >> 9, idx & 511], ones,
                                   mask=inb)
            return acc

        return lax.fori_loop(0, _CH // 16, vec_body, carry)

    lax.fori_loop(0, _PW // _CH, chunk_body, jnp.zeros((16,), jnp.int32))
    pltpu.sync_copy(histv, hist_out.at[wid])


def _select(hist, k, nrow, ncol):
    rowsum = jnp.sum(hist, axis=1, keepdims=True)
    i0 = lax.broadcasted_iota(jnp.int32, (nrow, nrow), 0)
    i1 = lax.broadcasted_iota(jnp.int32, (nrow, nrow), 1)
    m_rows = (i1 > i0).astype(jnp.float32)
    rows_above = lax.dot_general(m_rows, rowsum, (((1,), (0,)), ((), ())),
                                 preferred_element_type=jnp.float32)
    j0 = lax.broadcasted_iota(jnp.int32, (ncol, ncol), 0)
    j1 = lax.broadcasted_iota(jnp.int32, (ncol, ncol), 1)
    t_cols = (j0 > j1).astype(jnp.float32)
    row_suffix = lax.dot_general(hist, t_cols, (((1,), (0,)), ((), ())),
                                 preferred_element_type=jnp.float32)
    cum_above = rows_above + row_suffix
    sel = ((cum_above < k) & (cum_above + hist >= k)).astype(jnp.float32)
    br = lax.broadcasted_iota(jnp.int32, (nrow, ncol), 0).astype(jnp.float32)
    bc = lax.broadcasted_iota(jnp.int32, (nrow, ncol), 1).astype(jnp.float32)
    binf = jnp.sum(sel * (br * ncol + bc))
    c_above = jnp.sum(sel * cum_above)
    found = jnp.sum(sel)
    return binf, c_above, found


def _fold_bins(h_i32, ncol_in, groups):
    h = jnp.sum(h_i32.astype(jnp.float32), axis=0)
    g0 = lax.broadcasted_iota(jnp.int32, (ncol_in, groups), 0) >> 4
    g1 = lax.broadcasted_iota(jnp.int32, (ncol_in, groups), 1)
    fold = (g0 == g1).astype(jnp.float32)
    return lax.dot_general(h, fold, (((1,), (0,)), ((), ())),
                           preferred_element_type=jnp.float32)


def _getcol(selv, i):
    r0 = lax.broadcasted_iota(jnp.int32, (8, 128), 0) == 0
    ci = lax.broadcasted_iota(jnp.int32, (8, 128), 1) == i
    return jnp.sum(jnp.where(r0 & ci, selv, 0.0))


def _putrow0(vals):
    r0 = lax.broadcasted_iota(jnp.int32, (8, 128), 0) == 0
    col = lax.broadcasted_iota(jnp.int32, (8, 128), 1)
    out = jnp.zeros((8, 128), jnp.float32)
    for i, v in enumerate(vals):
        out = jnp.where(r0 & (col == i), v, out)
    return out


def _sel1_body(h_ref, p_ref, b1_ref, s_ref):
    bins = _fold_bins(h_ref[...], 256, 16)
    pos_cnt = jnp.sum(p_ref[...].astype(jnp.float32))
    kf = jnp.floor(pos_cnt * 0.5)
    binf, c_above, found = _select(bins, kf, 128, 16)
    b1_ref[...] = jnp.full((8, 128), binf.astype(jnp.int32))
    s_ref[...] = _putrow0([kf, pos_cnt, c_above, found, binf])


def _sel2_body(h_ref, s1_ref, s_ref):
    bins = _fold_bins(h_ref[...], 512, 32)
    s1 = s1_ref[...]
    kf = _getcol(s1, 0)
    pos_cnt = _getcol(s1, 1)
    c_above1 = _getcol(s1, 2)
    found1 = _getcol(s1, 3)
    b1f = _getcol(s1, 4)
    k2 = kf - c_above1
    b2f, c_above2, found2 = _select(bins, k2, 128, 32)
    valid = (found1 > 0.5) & (found2 > 0.5)
    s23 = (b1f - 1024.0) * 4096.0 + b2f
    extra = kf - c_above1 - c_above2
    kzero = kf < 0.5
    s23 = jnp.where(valid, s23,
                    jnp.where(kzero, 4194304.0, -4194305.0))
    extra = jnp.where(valid, extra, 0.0)
    s_ref[...] = _putrow0([s23, extra, kf, pos_cnt])


def _final_body(x_ref, t_ref, s_ref, o_ref, acc):
    pid = pl.program_id(0)

    @pl.when(pid == 0)
    def _init():
        acc[0] = 0.0
        acc[1] = 0.0
        o_ref[...] = jnp.zeros((8, 128), jnp.float32)

    sv = s_ref[...]
    s23 = _getcol(sv, 0)
    x = x_ref[...]
    t = t_ref[...]
    pos = t > 0
    neg = t == 0
    bce1 = jnp.maximum(x, 0.0) - x + jnp.log1p(jnp.exp(-jnp.abs(x)))
    acc[0] += jnp.sum(jnp.where(pos, bce1, 0.0))
    sk = _skey(x)
    sk23f = (sk >> 9).astype(jnp.float32)
    cond = neg & (sk23f > s23)
    acc[1] += jnp.sum(jnp.where(cond, _softplus(x), 0.0))

    @pl.when(pid == _GRID - 1)
    def _fin():
        extra = _getcol(sv, 1)
        kf = _getcol(sv, 2)
        pos_cnt = _getcol(sv, 3)
        s23c = jnp.clip(jnp.full((8, 128), s23), -4194304.0, 4194303.0)
        mid = s23c.astype(jnp.int32) * 512 + 256
        u = jnp.where(mid >= 0, mid, mid ^ jnp.int32(0x7FFFFFFF))
        v = lax.bitcast_convert_type(u, jnp.float32)
        sp_v = jnp.mean(_softplus(v))
        pos_sum = acc[0]
        neg_sum = acc[1] + extra * sp_v
        loss = (pos_sum + neg_sum) / (pos_cnt + kf)
        o_ref[...] = jnp.full((8, 128), loss)


def kernel(input, target):
    xf = input.reshape(-1)
    tf = target.reshape(-1).astype(jnp.int32)
    zz1 = jnp.zeros((128, 256), jnp.int32)
    zz2 = jnp.zeros((128, 512), jnp.int32)

    mesh = plsc.VectorSubcoreMesh(core_axis_name="c", subcore_axis_name="s")

    sc_l1 = pl.kernel(
        _sc_l1_body,
        out_type=[jax.ShapeDtypeStruct((_NW, 128, 256), jnp.int32),
                  jax.ShapeDtypeStruct((_NW, 16), jnp.int32)],
        mesh=mesh,
        scratch_types=[pltpu.VMEM((_CH,), jnp.float32),
                       pltpu.VMEM((_CH,), jnp.int32),
                       pltpu.VMEM((128, 256), jnp.int32),
                       pltpu.VMEM((16,), jnp.int32)],
        compiler_params=pltpu.CompilerParams(needs_layout_passes=False),
    )
    hist1, pos = sc_l1(xf, tf, zz1)

    sel1i, sel1f = pl.pallas_call(
        _sel1_body,
        grid=(1,),
        in_specs=[pl.BlockSpec((_NW, 128, 256), lambda i: (0, 0, 0)),
                  pl.BlockSpec((_NW, 16), lambda i: (0, 0))],
        out_specs=[pl.BlockSpec((8, 128), lambda i: (0, 0)),
                   pl.BlockSpec((8, 128), lambda i: (0, 0))],
        out_shape=[jax.ShapeDtypeStruct((8, 128), jnp.int32),
                   jax.ShapeDtypeStruct((8, 128), jnp.float32)],
    )(hist1, pos)

    sc_l2 = pl.kernel(
        _sc_l2_body,
        out_type=[jax.ShapeDtypeStruct((_NW, 128, 512), jnp.int32)],
        mesh=mesh,
        scratch_types=[pltpu.VMEM((_CH,), jnp.float32),
                       pltpu.VMEM((_CH,), jnp.int32),
                       pltpu.VMEM((128, 512), jnp.int32),
                       pltpu.VMEM((128,), jnp.int32)],
        compiler_params=pltpu.CompilerParams(needs_layout_passes=False),
    )
    (hist2,) = sc_l2(xf, tf, zz2, sel1i)

    sel2f = pl.pallas_call(
        _sel2_body,
        grid=(1,),
        in_specs=[pl.BlockSpec((_NW, 128, 512), lambda i: (0, 0, 0)),
                  pl.BlockSpec((8, 128), lambda i: (0, 0))],
        out_specs=pl.BlockSpec((8, 128), lambda i: (0, 0)),
        out_shape=jax.ShapeDtypeStruct((8, 128), jnp.float32),
    )(hist2, sel1f)

    xv = input.reshape(_R, _C)
    tv = tf.reshape(_R, _C)
    xspec = pl.BlockSpec((_BLK_R, _C), lambda i: (i, 0))
    sspec = pl.BlockSpec((8, 128), lambda i: (0, 0))
    out = pl.pallas_call(
        _final_body,
        grid=(_GRID,),
        in_specs=[xspec, xspec, sspec],
        out_specs=sspec,
        out_shape=jax.ShapeDtypeStruct((8, 128), jnp.float32),
        scratch_shapes=[pltpu.SMEM((2,), jnp.float32)],
    )(xv, tv, sel2f)

    return out[0, 0]
